# baseline (speedup 1.0000x reference)
import functools

import jax
import jax.numpy as jnp
from jax import lax
from jax.experimental import pallas as pl
from jax.experimental.pallas import tpu as pltpu
from jax.experimental.pallas import tpu_sc as plsc

NQ = 256
DM = 1024
BS = 8


@functools.cache
def _build_sc_broadcast():
    mesh = plsc.ScalarSubcoreMesh(axis_name="c", num_cores=2)
    bs_per_core = BS // 2

    @functools.partial(
        pl.kernel,
        mesh=mesh,
        out_type=jax.ShapeDtypeStruct((BS * NQ, DM), jnp.float32),
        scratch_types=[
            pltpu.MemorySpace.VMEM_SHARED((NQ, DM), jnp.float32),
            pltpu.SemaphoreType.DMA,
        ],
    )
    def broadcast_rows(qf_hbm, out_hbm, spmem, sem):
        cid = lax.axis_index("c")
        pltpu.sync_copy(qf_hbm, spmem)
        copies = [
            pltpu.async_copy(
                spmem, out_hbm.at[pl.ds((cid * bs_per_core + b) * NQ, NQ)], sem
            )
            for b in range(bs_per_core)
        ]
        for c in copies:
            c.wait()

    return broadcast_rows


def kernel(input_features, query_feat):
    bs = input_features.shape[1]
    out = _build_sc_broadcast()(query_feat)
    return out.reshape(bs, NQ, DM)
